# baseline (speedup 1.0000x reference)
import jax
import jax.numpy as jnp
from jax.experimental import pallas as pl

N = 10000


def _bn(h, g, b):
    mean = jnp.mean(h, axis=0)
    var = jnp.var(h, axis=0)
    return (h - mean) / jnp.sqrt(var + 1e-5) * g + b


def _identity_pallas(h):
    def body(i_ref, o_ref):
        o_ref[...] = i_ref[...]

    return pl.pallas_call(
        body,
        out_shape=jax.ShapeDtypeStruct(h.shape, h.dtype),
    )(h)


def kernel(x, edge_index, edge_weight, params):
    src = edge_index[0]
    dst = edge_index[1]
    w = edge_weight[:, None]

    def agg(h):
        return jax.ops.segment_sum(w * h[src], dst, num_segments=N)

    def gcn_layer(h, W, b, g, be):
        h = h @ W + b
        h = h + agg(h)
        return _bn(h, g, be)

    hg = x
    for l in (1, 2):
        hg = jax.nn.relu(gcn_layer(hg, params[f"gcn_W{l}"], params[f"gcn_b{l}"],
                                   params[f"gcn_g{l}"], params[f"gcn_be{l}"]))

    hi = x
    for l in (1, 2):
        neigh = agg(hi)
        hi = (1.0 + params["eps"][l - 1]) * hi + neigh
        t = _bn(hi @ params[f"gin{l}_A"] + params[f"gin{l}_a"],
                params[f"gin{l}_bg"], params[f"gin{l}_bb"])
        t = jax.nn.relu(t)
        hi = jax.nn.relu(t @ params[f"gin{l}_B"] + params[f"gin{l}_b2"])
        hi = jax.nn.relu(_bn(hi, params[f"gin{l}_og"], params[f"gin{l}_ob"]))

    h = jnp.concatenate([hg, hi], axis=1)
    for l in (1, 2):
        h = jax.nn.relu(gcn_layer(h, params[f"fin_W{l}"], params[f"fin_b{l}"],
                                  params[f"fin_g{l}"], params[f"fin_be{l}"]))
    out = h @ params["cls_W"] + params["cls_b"]
    return _identity_pallas(out)


# speedup vs baseline: 2.8997x; 2.8997x over previous
import dataclasses
import functools

import jax
import jax.numpy as jnp
from jax import lax
from jax.experimental import pallas as pl
from jax.experimental.pallas import tpu as pltpu
from jax.experimental.pallas import tpu_sc as plsc

NN = 10000
EE = 320000
HH = 128
NTILES = 16
CB = 128
EPT = 20480
EPAD = EPT * NTILES
NCHUNK = EPT // CB
RPT = 624
RPT_LAST = NN - 15 * RPT


def _sc_agg(hflat, src, dst, w):
    mesh = plsc.VectorSubcoreMesh(core_axis_name="c", subcore_axis_name="s")
    cp = pltpu.CompilerParams()
    if "needs_layout_passes" in pltpu.CompilerParams.__dataclass_fields__:
        cp = dataclasses.replace(cp, needs_layout_passes=False)

    @functools.partial(
        pl.kernel,
        mesh=mesh,
        compiler_params=cp,
        out_type=jax.ShapeDtypeStruct((2, NN, HH), jnp.float32),
        scratch_types=[
            pltpu.VMEM((CB,), jnp.int32),
            pltpu.VMEM((CB,), jnp.int32),
            pltpu.VMEM((CB,), jnp.float32),
            pltpu.VMEM((CB, HH), jnp.float32),
            pltpu.VMEM_SHARED((NN, HH), jnp.float32),
            pltpu.SemaphoreType.DMA,
        ],
    )
    def k(h_hbm, src_hbm, dst_hbm, w_hbm, out_hbm,
          srcv, dstv, wv, rows, acc, sem):
        cid = lax.axis_index("c")
        sid = lax.axis_index("s")

        zero16 = jnp.zeros((16,), jnp.float32)

        @pl.loop(0, CB)
        def _(r):
            for g in range(HH // 16):
                rows[r, pl.ds(g * 16, 16)] = zero16

        abase = pl.multiple_of(sid * RPT, 8)

        @pl.loop(0, 4)
        def _(z):
            pltpu.sync_copy(rows, acc.at[pl.ds(abase + z * CB, CB)])

        @pl.when(sid < NTILES - 1)
        def _():
            pltpu.sync_copy(rows.at[pl.ds(0, RPT - 4 * CB)],
                            acc.at[pl.ds(abase + 4 * CB, RPT - 4 * CB)])

        @pl.when(sid == NTILES - 1)
        def _():
            pltpu.sync_copy(rows, acc.at[pl.ds(15 * RPT + 4 * CB, CB)])

        plsc.subcore_barrier()

        off16 = jnp.full((16,), cid * NN, jnp.int32)

        @pl.loop(0, NCHUNK)
        def _(ck):
            base = pl.multiple_of(sid * EPT + ck * CB, 8)
            pltpu.sync_copy(src_hbm.at[pl.ds(base, CB)], srcv)
            pltpu.sync_copy(dst_hbm.at[pl.ds(base, CB)], dstv)
            pltpu.sync_copy(w_hbm.at[pl.ds(base, CB)], wv)
            for g in range(CB // 16):
                srcv[pl.ds(g * 16, 16)] = srcv[pl.ds(g * 16, 16)] + off16
            pltpu.async_copy(h_hbm.at[srcv], rows, sem).wait()

            @pl.loop(0, CB)
            def _(e):
                wj = plsc.load_gather(wv, [jnp.full((16,), e, jnp.int32)])
                for g in range(HH // 16):
                    rows[e, pl.ds(g * 16, 16)] = rows[e, pl.ds(g * 16, 16)] * wj

            pltpu.sync_copy(rows, acc.at[dstv], add=True)

        plsc.subcore_barrier()

        @pl.when(sid < NTILES - 1)
        def _():
            pltpu.sync_copy(acc.at[pl.ds(abase, RPT)],
                            out_hbm.at[cid, pl.ds(abase, RPT)])

        @pl.when(sid == NTILES - 1)
        def _():
            pltpu.sync_copy(acc.at[pl.ds(15 * RPT, RPT_LAST)],
                            out_hbm.at[cid, pl.ds(15 * RPT, RPT_LAST)])

    return k(hflat, src, dst, w)


def _bn_relu(h, g, be):
    mean = jnp.mean(h, axis=0, keepdims=True)
    var = jnp.mean((h - mean) ** 2, axis=0, keepdims=True)
    return jax.nn.relu((h - mean) / jnp.sqrt(var + 1e-5) * g + be)


def _dot(a, b):
    return jnp.dot(a, b, preferred_element_type=jnp.float32)


def _tc_call(body, out_shapes, *args):
    return pl.pallas_call(
        body,
        out_shape=out_shapes,
    )(*args)


def _stage_pre(x, W1, b1):
    def body(x_ref, w_ref, b_ref, o_ref):
        xv = x_ref[...]
        o_ref[0] = _dot(xv, w_ref[...]) + b_ref[...]
        o_ref[1] = xv
    return _tc_call(body, jax.ShapeDtypeStruct((2, NN, HH), jnp.float32),
                    x, W1, b1.reshape(1, HH))


def _stage_gcn(hpre, a, g, be, Wn, bn_):
    def body(hpre_ref, a_ref, g_ref, be_ref, w_ref, b_ref, o_ref):
        hg = _bn_relu(hpre_ref[...] + a_ref[...], g_ref[...], be_ref[...])
        o_ref[...] = _dot(hg, w_ref[...]) + b_ref[...]
    return _tc_call(body, jax.ShapeDtypeStruct((NN, HH), jnp.float32),
                    hpre, a, g.reshape(1, HH), be.reshape(1, HH),
                    Wn, bn_.reshape(1, HH))


def _stage_gcn_last(hpre, a, g, be):
    def body(hpre_ref, a_ref, g_ref, be_ref, o_ref):
        o_ref[...] = _bn_relu(hpre_ref[...] + a_ref[...], g_ref[...], be_ref[...])
    return _tc_call(body, jax.ShapeDtypeStruct((NN, HH), jnp.float32),
                    hpre, a, g.reshape(1, HH), be.reshape(1, HH))


def _stage_gin(hself, neigh, one_eps, A, a_, bg, bb, B, b2, og, ob):
    def body(h_ref, n_ref, e_ref, A_ref, a_ref, bg_ref, bb_ref,
             B_ref, b2_ref, og_ref, ob_ref, o_ref):
        hin = e_ref[...] * h_ref[...] + n_ref[...]
        t = _bn_relu(_dot(hin, A_ref[...]) + a_ref[...], bg_ref[...], bb_ref[...])
        h2 = jax.nn.relu(_dot(t, B_ref[...]) + b2_ref[...])
        o_ref[...] = _bn_relu(h2, og_ref[...], ob_ref[...])
    return _tc_call(body, jax.ShapeDtypeStruct((NN, HH), jnp.float32),
                    hself, neigh, one_eps.reshape(1, 1), A, a_.reshape(1, HH),
                    bg.reshape(1, HH), bb.reshape(1, HH), B, b2.reshape(1, HH),
                    og.reshape(1, HH), ob.reshape(1, HH))


def _stage_fin_pre(hg, hi, W, b):
    def body(hg_ref, hi_ref, w_ref, b_ref, o_ref):
        h = jnp.concatenate([hg_ref[...], hi_ref[...]], axis=1)
        f = _dot(h, w_ref[...]) + b_ref[...]
        o_ref[0] = f[:, :HH]
        o_ref[1] = f[:, HH:]
    return _tc_call(body, jax.ShapeDtypeStruct((2, NN, HH), jnp.float32),
                    hg, hi, W, b.reshape(1, 2 * HH))


def _stage_fin_mid(cat, agg, g, be, W, b):
    def body(cat_ref, agg_ref, g_ref, be_ref, w_ref, b_ref, o_ref):
        fpre = jnp.concatenate([cat_ref[0], cat_ref[1]], axis=1)
        a = jnp.concatenate([agg_ref[0], agg_ref[1]], axis=1)
        f = _bn_relu(fpre + a, g_ref[...], be_ref[...])
        o = _dot(f, w_ref[...]) + b_ref[...]
        o_ref[0] = o[:, :HH]
        o_ref[1] = o[:, HH:]
    return _tc_call(body, jax.ShapeDtypeStruct((2, NN, HH), jnp.float32),
                    cat, agg, g.reshape(1, 2 * HH), be.reshape(1, 2 * HH),
                    W, b.reshape(1, 2 * HH))


def _stage_out(cat, agg, g, be, W, b):
    def body(cat_ref, agg_ref, g_ref, be_ref, w_ref, b_ref, o_ref):
        fpre = jnp.concatenate([cat_ref[0], cat_ref[1]], axis=1)
        a = jnp.concatenate([agg_ref[0], agg_ref[1]], axis=1)
        f = _bn_relu(fpre + a, g_ref[...], be_ref[...])
        o_ref[...] = _dot(f, w_ref[...]) + b_ref[...]
    return _tc_call(body, jax.ShapeDtypeStruct((NN, 64), jnp.float32),
                    cat, agg, g.reshape(1, 2 * HH), be.reshape(1, 2 * HH),
                    W, b.reshape(1, 64))


def kernel(x, edge_index, edge_weight, params):
    p = params
    src = edge_index[0]
    dst = edge_index[1]

    npad = EPAD - EE
    padidx = (jnp.arange(npad, dtype=jnp.int32) * 13) % NN
    srcp = jnp.concatenate([src, padidx])
    dstp = jnp.concatenate([dst, padidx])
    wp = jnp.concatenate([edge_weight, jnp.zeros((npad,), jnp.float32)])

    one_eps = 1.0 + p["eps"]

    cat0 = _stage_pre(x, p["gcn_W1"], p["gcn_b1"])
    agg1 = _sc_agg(cat0.reshape(2 * NN, HH), srcp, dstp, wp)

    hg2_pre = _stage_gcn(cat0[0], agg1[0], p["gcn_g1"], p["gcn_be1"],
                         p["gcn_W2"], p["gcn_b2"])
    hi1 = _stage_gin(cat0[1], agg1[1], one_eps[0:1], p["gin1_A"], p["gin1_a"],
                     p["gin1_bg"], p["gin1_bb"], p["gin1_B"], p["gin1_b2"],
                     p["gin1_og"], p["gin1_ob"])
    cat1 = jnp.stack([hg2_pre, hi1])
    agg2 = _sc_agg(cat1.reshape(2 * NN, HH), srcp, dstp, wp)

    hg2 = _stage_gcn_last(cat1[0], agg2[0], p["gcn_g2"], p["gcn_be2"])
    hi2 = _stage_gin(cat1[1], agg2[1], one_eps[1:2], p["gin2_A"], p["gin2_a"],
                     p["gin2_bg"], p["gin2_bb"], p["gin2_B"], p["gin2_b2"],
                     p["gin2_og"], p["gin2_ob"])

    cat2 = _stage_fin_pre(hg2, hi2, p["fin_W1"], p["fin_b1"])
    agg3 = _sc_agg(cat2.reshape(2 * NN, HH), srcp, dstp, wp)

    cat3 = _stage_fin_mid(cat2, agg3, p["fin_g1"], p["fin_be1"],
                          p["fin_W2"], p["fin_b2"])
    agg4 = _sc_agg(cat3.reshape(2 * NN, HH), srcp, dstp, wp)

    return _stage_out(cat3, agg4, p["fin_g2"], p["fin_be2"],
                      p["cls_W"], p["cls_b"])


# speedup vs baseline: 8.4846x; 2.9261x over previous
import dataclasses
import functools

import jax
import jax.numpy as jnp
from jax import lax
from jax.experimental import pallas as pl
from jax.experimental.pallas import tpu as pltpu
from jax.experimental.pallas import tpu_sc as plsc

NN = 10000
EE = 320000
HH = 128
NTILES = 16
CB = 112
NCHUNK = 180
EPT = NCHUNK * CB
EPAD = EPT * NTILES
RPT = 624
RPT_LAST = NN - 15 * RPT


def _sc_agg(hflat, edata):
    mesh = plsc.VectorSubcoreMesh(core_axis_name="c", subcore_axis_name="s")
    cp = pltpu.CompilerParams()
    if "needs_layout_passes" in pltpu.CompilerParams.__dataclass_fields__:
        cp = dataclasses.replace(cp, needs_layout_passes=False)

    @functools.partial(
        pl.kernel,
        mesh=mesh,
        compiler_params=cp,
        out_type=jax.ShapeDtypeStruct((2, NN, HH), jnp.float32),
        scratch_types=(
            [pltpu.VMEM((3, CB), jnp.int32)] * 3
            + [pltpu.VMEM((CB,), jnp.int32)] * 3
            + [pltpu.VMEM((CB, HH), jnp.float32)] * 3
            + [pltpu.VMEM_SHARED((NN, HH), jnp.float32)]
            + [pltpu.SemaphoreType.DMA] * 9
        ),
    )
    def k(h_hbm, e_hbm, out_hbm,
          eb0, eb1, eb2, dv0, dv1, dv2, rw0, rw1, rw2, acc,
          es0, es1, es2, gs0, gs1, gs2, ss0, ss1, ss2):
        cid = lax.axis_index("c")
        sid = lax.axis_index("s")
        ebs = (eb0, eb1, eb2)
        dvs = (dv0, dv1, dv2)
        rws = (rw0, rw1, rw2)
        ess = (es0, es1, es2)
        gss = (gs0, gs1, gs2)
        sss = (ss0, ss1, ss2)

        zero16 = jnp.zeros((16,), jnp.float32)
        off16 = jnp.full((16,), cid * NN, jnp.int32)

        @pl.loop(0, CB)
        def _(r):
            for g in range(HH // 16):
                rw0[r, pl.ds(g * 16, 16)] = zero16

        abase = pl.multiple_of(sid * RPT, 8)

        @pl.loop(0, 5)
        def _(z):
            pltpu.sync_copy(rw0, acc.at[pl.ds(abase + z * CB, CB)])

        @pl.when(sid < NTILES - 1)
        def _():
            pltpu.sync_copy(rw0.at[pl.ds(0, RPT - 5 * CB)],
                            acc.at[pl.ds(abase + 5 * CB, RPT - 5 * CB)])

        @pl.when(sid == NTILES - 1)
        def _():
            pltpu.sync_copy(rw0.at[pl.ds(0, RPT_LAST - 5 * CB)],
                            acc.at[pl.ds(15 * RPT + 5 * CB, RPT_LAST - 5 * CB)])

        plsc.subcore_barrier()

        def eload(c, j):
            pltpu.async_copy(e_hbm.at[sid, c], ebs[j], ess[j])

        def ewait(j):
            pltpu.make_async_copy(e_hbm.at[sid, 0], ebs[j], ess[j]).wait()

        def offadd(j):
            for g in range(CB // 16):
                ebs[j][0, pl.ds(g * 16, 16)] = (
                    ebs[j][0, pl.ds(g * 16, 16)] + off16)

        def gstart(j):
            pltpu.async_copy(h_hbm.at[ebs[j].at[0]], rws[j], gss[j])

        def gwait(j):
            pltpu.make_async_copy(h_hbm.at[ebs[j].at[0]], rws[j], gss[j]).wait()

        def mult(j):
            for g in range(CB // 16):
                dvs[j][pl.ds(g * 16, 16)] = ebs[j][1, pl.ds(g * 16, 16)]

            @plsc.parallel_loop(0, CB, unroll=4)
            def _(e):
                wj = plsc.bitcast(
                    plsc.load_gather(ebs[j].at[2],
                                     [jnp.full((16,), e, jnp.int32)]),
                    jnp.float32)
                for g in range(HH // 16):
                    rws[j][e, pl.ds(g * 16, 16)] = (
                        rws[j][e, pl.ds(g * 16, 16)] * wj)

        def sstart(j):
            pltpu.async_copy(rws[j], acc.at[dvs[j]], sss[j], add=True)

        def swait(j):
            pltpu.make_async_copy(rws[j], acc.at[dvs[j]], sss[j]).wait()

        for j in range(2):
            pltpu.sync_copy(e_hbm.at[sid, j], ebs[j])
            offadd(j)
            gstart(j)
        eload(2, 2)

        @pl.loop(0, NCHUNK, step=3)
        def _(ck):
            @pl.when(ck > 0)
            def _():
                swait(2)
            ewait(2)
            offadd(2)
            gstart(2)

            gwait(0)
            mult(0)
            sstart(0)

            @pl.when(ck + 3 < NCHUNK)
            def _():
                eload(ck + 3, 0)

            gwait(1)
            mult(1)
            sstart(1)

            @pl.when(ck + 4 < NCHUNK)
            def _():
                eload(ck + 4, 1)

            @pl.when(ck + 3 < NCHUNK)
            def _():
                swait(0)
                ewait(0)
                offadd(0)
                gstart(0)

            gwait(2)
            mult(2)
            sstart(2)

            @pl.when(ck + 5 < NCHUNK)
            def _():
                eload(ck + 5, 2)

            @pl.when(ck + 4 < NCHUNK)
            def _():
                swait(1)
                ewait(1)
                offadd(1)
                gstart(1)

        swait(0)
        swait(1)
        swait(2)
        plsc.subcore_barrier()

        @pl.when(sid < NTILES - 1)
        def _():
            pltpu.sync_copy(acc.at[pl.ds(abase, RPT)],
                            out_hbm.at[cid, pl.ds(abase, RPT)])

        @pl.when(sid == NTILES - 1)
        def _():
            pltpu.sync_copy(acc.at[pl.ds(15 * RPT, RPT_LAST)],
                            out_hbm.at[cid, pl.ds(15 * RPT, RPT_LAST)])

    return k(hflat, edata)


def _bn_relu(h, g, be):
    mean = jnp.mean(h, axis=0, keepdims=True)
    var = jnp.mean((h - mean) ** 2, axis=0, keepdims=True)
    return jax.nn.relu((h - mean) / jnp.sqrt(var + 1e-5) * g + be)


def _dot(a, b):
    return jnp.dot(a, b, preferred_element_type=jnp.float32)


def _tc_call(body, out_shapes, *args):
    return pl.pallas_call(
        body,
        out_shape=out_shapes,
    )(*args)


def _stage_pre(x, W1, b1):
    def body(x_ref, w_ref, b_ref, o_ref):
        xv = x_ref[...]
        o_ref[0] = _dot(xv, w_ref[...]) + b_ref[...]
        o_ref[1] = xv
    return _tc_call(body, jax.ShapeDtypeStruct((2, NN, HH), jnp.float32),
                    x, W1, b1.reshape(1, HH))


def _stage_gcn(hpre, a, g, be, Wn, bn_):
    def body(hpre_ref, a_ref, g_ref, be_ref, w_ref, b_ref, o_ref):
        hg = _bn_relu(hpre_ref[...] + a_ref[...], g_ref[...], be_ref[...])
        o_ref[...] = _dot(hg, w_ref[...]) + b_ref[...]
    return _tc_call(body, jax.ShapeDtypeStruct((NN, HH), jnp.float32),
                    hpre, a, g.reshape(1, HH), be.reshape(1, HH),
                    Wn, bn_.reshape(1, HH))


def _stage_gcn_last(hpre, a, g, be):
    def body(hpre_ref, a_ref, g_ref, be_ref, o_ref):
        o_ref[...] = _bn_relu(hpre_ref[...] + a_ref[...], g_ref[...], be_ref[...])
    return _tc_call(body, jax.ShapeDtypeStruct((NN, HH), jnp.float32),
                    hpre, a, g.reshape(1, HH), be.reshape(1, HH))


def _stage_gin(hself, neigh, one_eps, A, a_, bg, bb, B, b2, og, ob):
    def body(h_ref, n_ref, e_ref, A_ref, a_ref, bg_ref, bb_ref,
             B_ref, b2_ref, og_ref, ob_ref, o_ref):
        hin = e_ref[...] * h_ref[...] + n_ref[...]
        t = _bn_relu(_dot(hin, A_ref[...]) + a_ref[...], bg_ref[...], bb_ref[...])
        h2 = jax.nn.relu(_dot(t, B_ref[...]) + b2_ref[...])
        o_ref[...] = _bn_relu(h2, og_ref[...], ob_ref[...])
    return _tc_call(body, jax.ShapeDtypeStruct((NN, HH), jnp.float32),
                    hself, neigh, one_eps.reshape(1, 1), A, a_.reshape(1, HH),
                    bg.reshape(1, HH), bb.reshape(1, HH), B, b2.reshape(1, HH),
                    og.reshape(1, HH), ob.reshape(1, HH))


def _stage_fin_pre(hg, hi, W, b):
    def body(hg_ref, hi_ref, w_ref, b_ref, o_ref):
        h = jnp.concatenate([hg_ref[...], hi_ref[...]], axis=1)
        f = _dot(h, w_ref[...]) + b_ref[...]
        o_ref[0] = f[:, :HH]
        o_ref[1] = f[:, HH:]
    return _tc_call(body, jax.ShapeDtypeStruct((2, NN, HH), jnp.float32),
                    hg, hi, W, b.reshape(1, 2 * HH))


def _stage_fin_mid(cat, agg, g, be, W, b):
    def body(cat_ref, agg_ref, g_ref, be_ref, w_ref, b_ref, o_ref):
        fpre = jnp.concatenate([cat_ref[0], cat_ref[1]], axis=1)
        a = jnp.concatenate([agg_ref[0], agg_ref[1]], axis=1)
        f = _bn_relu(fpre + a, g_ref[...], be_ref[...])
        o = _dot(f, w_ref[...]) + b_ref[...]
        o_ref[0] = o[:, :HH]
        o_ref[1] = o[:, HH:]
    return _tc_call(body, jax.ShapeDtypeStruct((2, NN, HH), jnp.float32),
                    cat, agg, g.reshape(1, 2 * HH), be.reshape(1, 2 * HH),
                    W, b.reshape(1, 2 * HH))


def _stage_out(cat, agg, g, be, W, b):
    def body(cat_ref, agg_ref, g_ref, be_ref, w_ref, b_ref, o_ref):
        fpre = jnp.concatenate([cat_ref[0], cat_ref[1]], axis=1)
        a = jnp.concatenate([agg_ref[0], agg_ref[1]], axis=1)
        f = _bn_relu(fpre + a, g_ref[...], be_ref[...])
        o_ref[...] = _dot(f, w_ref[...]) + b_ref[...]
    return _tc_call(body, jax.ShapeDtypeStruct((NN, 64), jnp.float32),
                    cat, agg, g.reshape(1, 2 * HH), be.reshape(1, 2 * HH),
                    W, b.reshape(1, 64))


def kernel(x, edge_index, edge_weight, params):
    p = params
    src = edge_index[0]
    dst = edge_index[1]

    npad = EPAD - EE
    padidx = (jnp.arange(npad, dtype=jnp.int32) * 13) % NN
    srcp = jnp.concatenate([src, padidx])
    dstp = jnp.concatenate([dst, padidx])
    wp = jnp.concatenate([edge_weight, jnp.zeros((npad,), jnp.float32)])
    wbits = lax.bitcast_convert_type(wp, jnp.int32)
    edata = (jnp.stack([srcp, dstp, wbits])
             .reshape(3, NTILES, NCHUNK, CB)
             .transpose(1, 2, 0, 3))

    one_eps = 1.0 + p["eps"]

    cat0 = _stage_pre(x, p["gcn_W1"], p["gcn_b1"])
    agg1 = _sc_agg(cat0.reshape(2 * NN, HH), edata)

    hg2_pre = _stage_gcn(cat0[0], agg1[0], p["gcn_g1"], p["gcn_be1"],
                         p["gcn_W2"], p["gcn_b2"])
    hi1 = _stage_gin(cat0[1], agg1[1], one_eps[0:1], p["gin1_A"], p["gin1_a"],
                     p["gin1_bg"], p["gin1_bb"], p["gin1_B"], p["gin1_b2"],
                     p["gin1_og"], p["gin1_ob"])
    cat1 = jnp.stack([hg2_pre, hi1])
    agg2 = _sc_agg(cat1.reshape(2 * NN, HH), edata)

    hg2 = _stage_gcn_last(cat1[0], agg2[0], p["gcn_g2"], p["gcn_be2"])
    hi2 = _stage_gin(cat1[1], agg2[1], one_eps[1:2], p["gin2_A"], p["gin2_a"],
                     p["gin2_bg"], p["gin2_bb"], p["gin2_B"], p["gin2_b2"],
                     p["gin2_og"], p["gin2_ob"])

    cat2 = _stage_fin_pre(hg2, hi2, p["fin_W1"], p["fin_b1"])
    agg3 = _sc_agg(cat2.reshape(2 * NN, HH), edata)

    cat3 = _stage_fin_mid(cat2, agg3, p["fin_g1"], p["fin_be1"],
                          p["fin_W2"], p["fin_b2"])
    agg4 = _sc_agg(cat3.reshape(2 * NN, HH), edata)

    return _stage_out(cat3, agg4, p["fin_g2"], p["fin_be2"],
                      p["cls_W"], p["cls_b"])


# speedup vs baseline: 8.6417x; 1.0185x over previous
import dataclasses
import functools

import jax
import jax.numpy as jnp
from jax import lax
from jax.experimental import pallas as pl
from jax.experimental.pallas import tpu as pltpu
from jax.experimental.pallas import tpu_sc as plsc

NN = 10000
EE = 320000
HH = 128
NTILES = 16
CB = 112
NCHUNK = 180
EPT = NCHUNK * CB
EPAD = EPT * NTILES
RPT = 624
RPT_LAST = NN - 15 * RPT


def _sc_agg(hflat, edata):
    mesh = plsc.VectorSubcoreMesh(core_axis_name="c", subcore_axis_name="s")
    cp = pltpu.CompilerParams()
    if "needs_layout_passes" in pltpu.CompilerParams.__dataclass_fields__:
        cp = dataclasses.replace(cp, needs_layout_passes=False)

    @functools.partial(
        pl.kernel,
        mesh=mesh,
        compiler_params=cp,
        out_type=jax.ShapeDtypeStruct((2, NN, HH), jnp.float32),
        scratch_types=(
            [pltpu.VMEM((3, CB), jnp.int32)] * 3
            + [pltpu.VMEM((CB,), jnp.int32)] * 3
            + [pltpu.VMEM((CB, HH), jnp.float32)] * 3
            + [pltpu.VMEM_SHARED((NN, HH), jnp.float32)]
            + [pltpu.SemaphoreType.DMA] * 9
        ),
    )
    def k(h_hbm, e_hbm, out_hbm,
          eb0, eb1, eb2, dv0, dv1, dv2, rw0, rw1, rw2, acc,
          es0, es1, es2, gs0, gs1, gs2, ss0, ss1, ss2):
        cid = lax.axis_index("c")
        sid = lax.axis_index("s")
        ebs = (eb0, eb1, eb2)
        dvs = (dv0, dv1, dv2)
        rws = (rw0, rw1, rw2)
        ess = (es0, es1, es2)
        gss = (gs0, gs1, gs2)
        sss = (ss0, ss1, ss2)

        zero16 = jnp.zeros((16,), jnp.float32)
        off16 = jnp.full((16,), cid * NN, jnp.int32)

        @pl.loop(0, CB)
        def _(r):
            for g in range(HH // 16):
                rw0[r, pl.ds(g * 16, 16)] = zero16

        abase = pl.multiple_of(sid * RPT, 8)

        @pl.loop(0, 5)
        def _(z):
            pltpu.sync_copy(rw0, acc.at[pl.ds(abase + z * CB, CB)])

        @pl.when(sid < NTILES - 1)
        def _():
            pltpu.sync_copy(rw0.at[pl.ds(0, RPT - 5 * CB)],
                            acc.at[pl.ds(abase + 5 * CB, RPT - 5 * CB)])

        @pl.when(sid == NTILES - 1)
        def _():
            pltpu.sync_copy(rw0.at[pl.ds(0, RPT_LAST - 5 * CB)],
                            acc.at[pl.ds(15 * RPT + 5 * CB, RPT_LAST - 5 * CB)])

        plsc.subcore_barrier()

        def eload(c, j):
            pltpu.async_copy(e_hbm.at[sid, c], ebs[j], ess[j])

        def ewait(j):
            pltpu.make_async_copy(e_hbm.at[sid, 0], ebs[j], ess[j]).wait()

        def offadd(j):
            for g in range(CB // 16):
                ebs[j][0, pl.ds(g * 16, 16)] = (
                    ebs[j][0, pl.ds(g * 16, 16)] + off16)

        def gstart(j):
            pltpu.async_copy(h_hbm.at[ebs[j].at[0]], rws[j], gss[j])

        def gwait(j):
            pltpu.make_async_copy(h_hbm.at[ebs[j].at[0]], rws[j], gss[j]).wait()

        def mult(j):
            for g in range(CB // 16):
                dvs[j][pl.ds(g * 16, 16)] = ebs[j][1, pl.ds(g * 16, 16)]

            @plsc.parallel_loop(0, CB, unroll=4)
            def _(e):
                wj = plsc.bitcast(
                    plsc.load_gather(ebs[j].at[2],
                                     [jnp.full((16,), e, jnp.int32)]),
                    jnp.float32)
                for g in range(HH // 16):
                    rws[j][e, pl.ds(g * 16, 16)] = (
                        rws[j][e, pl.ds(g * 16, 16)] * wj)

        def sstart(j):
            pltpu.async_copy(rws[j], acc.at[dvs[j]], sss[j], add=True)

        def swait(j):
            pltpu.make_async_copy(rws[j], acc.at[dvs[j]], sss[j]).wait()

        for j in range(2):
            pltpu.sync_copy(e_hbm.at[sid, j], ebs[j])
            offadd(j)
            gstart(j)
        eload(2, 2)

        @pl.loop(0, NCHUNK, step=3)
        def _(ck):
            @pl.when(ck > 0)
            def _():
                swait(2)
            ewait(2)
            offadd(2)
            gstart(2)

            gwait(0)
            mult(0)
            sstart(0)

            @pl.when(ck + 3 < NCHUNK)
            def _():
                eload(ck + 3, 0)

            gwait(1)
            mult(1)
            sstart(1)

            @pl.when(ck + 4 < NCHUNK)
            def _():
                eload(ck + 4, 1)

            @pl.when(ck + 3 < NCHUNK)
            def _():
                swait(0)
                ewait(0)
                offadd(0)
                gstart(0)

            gwait(2)
            mult(2)
            sstart(2)

            @pl.when(ck + 5 < NCHUNK)
            def _():
                eload(ck + 5, 2)

            @pl.when(ck + 4 < NCHUNK)
            def _():
                swait(1)
                ewait(1)
                offadd(1)
                gstart(1)

        swait(0)
        swait(1)
        swait(2)
        plsc.subcore_barrier()

        @pl.when(sid < NTILES - 1)
        def _():
            pltpu.sync_copy(acc.at[pl.ds(abase, RPT)],
                            out_hbm.at[cid, pl.ds(abase, RPT)])

        @pl.when(sid == NTILES - 1)
        def _():
            pltpu.sync_copy(acc.at[pl.ds(15 * RPT, RPT_LAST)],
                            out_hbm.at[cid, pl.ds(15 * RPT, RPT_LAST)])

    return k(hflat, edata)


def _bn_relu(h, g, be):
    mean = jnp.mean(h, axis=0, keepdims=True)
    var = jnp.mean((h - mean) ** 2, axis=0, keepdims=True)
    return jax.nn.relu((h - mean) / jnp.sqrt(var + 1e-5) * g + be)


def _dot(a, b):
    return jnp.dot(a, b, preferred_element_type=jnp.float32)


def _tc_call(body, out_shapes, *args):
    return pl.pallas_call(
        body,
        out_shape=out_shapes,
    )(*args)


def _stage_pre(x, W1, b1):
    def body(x_ref, w_ref, b_ref, o_ref):
        xv = x_ref[...]
        o_ref[0] = _dot(xv, w_ref[...]) + b_ref[...]
        o_ref[1] = xv
    return _tc_call(body, jax.ShapeDtypeStruct((2, NN, HH), jnp.float32),
                    x, W1, b1.reshape(1, HH))


def _stage_b(cat0, agg1, one_eps, g1, be1, W2, b2, A, a_, bg, bb, B, b2g, og, ob):
    def body(cat_ref, agg_ref, e_ref, g1_ref, be1_ref, w2_ref, b2_ref,
             A_ref, a_ref, bg_ref, bb_ref, B_ref, b2g_ref, og_ref, ob_ref,
             o_ref):
        hg1 = _bn_relu(cat_ref[0] + agg_ref[0], g1_ref[...], be1_ref[...])
        o_ref[0] = _dot(hg1, w2_ref[...]) + b2_ref[...]
        hin = e_ref[...] * cat_ref[1] + agg_ref[1]
        t = _bn_relu(_dot(hin, A_ref[...]) + a_ref[...], bg_ref[...], bb_ref[...])
        h2 = jax.nn.relu(_dot(t, B_ref[...]) + b2g_ref[...])
        o_ref[1] = _bn_relu(h2, og_ref[...], ob_ref[...])
    return _tc_call(body, jax.ShapeDtypeStruct((2, NN, HH), jnp.float32),
                    cat0, agg1, one_eps.reshape(1, 1), g1.reshape(1, HH),
                    be1.reshape(1, HH), W2, b2.reshape(1, HH), A,
                    a_.reshape(1, HH), bg.reshape(1, HH), bb.reshape(1, HH),
                    B, b2g.reshape(1, HH), og.reshape(1, HH), ob.reshape(1, HH))


def _stage_c1(cat1, agg2, one_eps, g2, be2, A, a_, bg, bb, B, b2g, og, ob):
    def body(cat_ref, agg_ref, e_ref, g2_ref, be2_ref,
             A_ref, a_ref, bg_ref, bb_ref, B_ref, b2g_ref, og_ref, ob_ref,
             o_ref):
        o_ref[0] = _bn_relu(cat_ref[0] + agg_ref[0], g2_ref[...], be2_ref[...])
        hin = e_ref[...] * cat_ref[1] + agg_ref[1]
        t = _bn_relu(_dot(hin, A_ref[...]) + a_ref[...], bg_ref[...], bb_ref[...])
        h2 = jax.nn.relu(_dot(t, B_ref[...]) + b2g_ref[...])
        o_ref[1] = _bn_relu(h2, og_ref[...], ob_ref[...])
    return _tc_call(body, jax.ShapeDtypeStruct((2, NN, HH), jnp.float32),
                    cat1, agg2, one_eps.reshape(1, 1), g2.reshape(1, HH),
                    be2.reshape(1, HH), A, a_.reshape(1, HH), bg.reshape(1, HH),
                    bb.reshape(1, HH), B, b2g.reshape(1, HH), og.reshape(1, HH),
                    ob.reshape(1, HH))


def _stage_c2(catgh, W, b):
    def body(cat_ref, w_ref, b_ref, o_ref):
        f = (_dot(cat_ref[0], w_ref[pl.ds(0, HH), :])
             + _dot(cat_ref[1], w_ref[pl.ds(HH, HH), :]) + b_ref[...])
        o_ref[0] = f[:, :HH]
        o_ref[1] = f[:, HH:]
    return _tc_call(body, jax.ShapeDtypeStruct((2, NN, HH), jnp.float32),
                    catgh, W, b.reshape(1, 2 * HH))


def _stage_d(cat, agg, g, be, W, b):
    def body(cat_ref, agg_ref, g_ref, be_ref, w_ref, b_ref, o_ref):
        f0 = _bn_relu(cat_ref[0] + agg_ref[0], g_ref[:, :HH], be_ref[:, :HH])
        f1 = _bn_relu(cat_ref[1] + agg_ref[1], g_ref[:, HH:], be_ref[:, HH:])
        o = (_dot(f0, w_ref[pl.ds(0, HH), :]) + _dot(f1, w_ref[pl.ds(HH, HH), :])
             + b_ref[...])
        o_ref[0] = o[:, :HH]
        o_ref[1] = o[:, HH:]
    return _tc_call(body, jax.ShapeDtypeStruct((2, NN, HH), jnp.float32),
                    cat, agg, g.reshape(1, 2 * HH), be.reshape(1, 2 * HH),
                    W, b.reshape(1, 2 * HH))


def _stage_out(cat, agg, g, be, W, b):
    def body(cat_ref, agg_ref, g_ref, be_ref, w_ref, b_ref, o_ref):
        f0 = _bn_relu(cat_ref[0] + agg_ref[0], g_ref[:, :HH], be_ref[:, :HH])
        f1 = _bn_relu(cat_ref[1] + agg_ref[1], g_ref[:, HH:], be_ref[:, HH:])
        o_ref[...] = (_dot(f0, w_ref[pl.ds(0, HH), :])
                      + _dot(f1, w_ref[pl.ds(HH, HH), :]) + b_ref[...])
    return _tc_call(body, jax.ShapeDtypeStruct((NN, 64), jnp.float32),
                    cat, agg, g.reshape(1, 2 * HH), be.reshape(1, 2 * HH),
                    W, b.reshape(1, 64))


def kernel(x, edge_index, edge_weight, params):
    p = params
    src = edge_index[0]
    dst = edge_index[1]

    npad = EPAD - EE
    padidx = (jnp.arange(npad, dtype=jnp.int32) * 13) % NN
    srcp = jnp.concatenate([src, padidx])
    dstp = jnp.concatenate([dst, padidx])
    wp = jnp.concatenate([edge_weight, jnp.zeros((npad,), jnp.float32)])
    wbits = lax.bitcast_convert_type(wp, jnp.int32)
    edata = (jnp.stack([srcp, dstp, wbits])
             .reshape(3, NTILES, NCHUNK, CB)
             .transpose(1, 2, 0, 3))

    one_eps = 1.0 + p["eps"]

    cat0 = _stage_pre(x, p["gcn_W1"], p["gcn_b1"])
    agg1 = _sc_agg(cat0.reshape(2 * NN, HH), edata)

    cat1 = _stage_b(cat0, agg1, one_eps[0:1], p["gcn_g1"], p["gcn_be1"],
                    p["gcn_W2"], p["gcn_b2"], p["gin1_A"], p["gin1_a"],
                    p["gin1_bg"], p["gin1_bb"], p["gin1_B"], p["gin1_b2"],
                    p["gin1_og"], p["gin1_ob"])
    agg2 = _sc_agg(cat1.reshape(2 * NN, HH), edata)

    catgh = _stage_c1(cat1, agg2, one_eps[1:2], p["gcn_g2"], p["gcn_be2"],
                      p["gin2_A"], p["gin2_a"], p["gin2_bg"], p["gin2_bb"],
                      p["gin2_B"], p["gin2_b2"], p["gin2_og"], p["gin2_ob"])

    cat2 = _stage_c2(catgh, p["fin_W1"], p["fin_b1"])
    agg3 = _sc_agg(cat2.reshape(2 * NN, HH), edata)

    cat3 = _stage_d(cat2, agg3, p["fin_g1"], p["fin_be1"],
                    p["fin_W2"], p["fin_b2"])
    agg4 = _sc_agg(cat3.reshape(2 * NN, HH), edata)

    return _stage_out(cat3, agg4, p["fin_g2"], p["fin_be2"],
                      p["cls_W"], p["cls_b"])
